# Initial kernel scaffold; baseline (speedup 1.0000x reference)
import jax
import jax.numpy as jnp
from jax.experimental import pallas as pl


def kernel(inputs):
    raise NotImplementedError("write your pallas kernel here")


# speedup vs baseline: 7.8280x; 7.8280x over previous
import functools

import jax
import jax.numpy as jnp
from jax import lax
from jax.experimental import pallas as pl
from jax.experimental.pallas import tpu as pltpu

K = 20
N = 2048
ROW_BLOCK = 256


def _topk_body(x_ref, o_ref):
    x = x_ref[...]
    r = x.shape[0]
    iota = lax.broadcasted_iota(jnp.int32, (r, N), 1)
    cols = []
    for t in range(K + 1):
        m = jnp.min(x, axis=1, keepdims=True)
        idx = jnp.min(jnp.where(x == m, iota, N), axis=1, keepdims=True)
        if t > 0:
            cols.append(idx)
        x = jnp.where(iota == idx, jnp.float32(jnp.inf), x)
    o_ref[...] = jnp.concatenate(cols, axis=1)


@jax.jit
def kernel(inputs):
    d = inputs
    b, q, n = d.shape
    rows = d.reshape(b * q, n)
    out = pl.pallas_call(
        _topk_body,
        grid=(b * q // ROW_BLOCK,),
        in_specs=[pl.BlockSpec((ROW_BLOCK, N), lambda i: (i, 0))],
        out_specs=pl.BlockSpec((ROW_BLOCK, K), lambda i: (i, 0)),
        out_shape=jax.ShapeDtypeStruct((b * q, K), jnp.int32),
        compiler_params=pltpu.CompilerParams(
            dimension_semantics=("arbitrary",),
        ),
    )(rows)
    return out.reshape(b, q, K)
